# Initial kernel scaffold; baseline (speedup 1.0000x reference)
import jax
import jax.numpy as jnp
from jax.experimental import pallas as pl


def kernel(x, rms_scale, in_proj_w, in_proj_b, out_proj_w, out_proj_b, router_w, router_b, w1, b1, w2, b2):
    raise NotImplementedError("write your pallas kernel here")


# speedup vs baseline: 1.3638x; 1.3638x over previous
import functools

import jax
import jax.numpy as jnp
import numpy as np
from jax.experimental import pallas as pl
from jax.experimental.pallas import tpu as pltpu

B, T, D = 1, 2048, 1024
H = 16
HD = D // H
E = 8
TOPK = 2
DFF = 4 * D
EPS = 1e-6
WIN = T // 2
BT = 256
KW = WIN + BT


def _rope_tables():
    hd = HD
    theta = 1.0 / (10000.0 ** (np.arange(0, hd, 2, dtype=np.float64) / hd))
    idx = np.arange(T, dtype=np.float64)[:, None] * theta[None, :]
    cos = np.cos(idx)
    sin = np.sin(idx)
    cos_full = np.repeat(cos, 2, axis=1)
    sin_full = np.repeat(sin, 2, axis=1)
    cos_full = np.tile(cos_full, (1, H))
    sin_full = np.tile(sin_full, (1, H))
    return (jnp.asarray(cos_full, jnp.float32), jnp.asarray(sin_full, jnp.float32))


def _qkv_kernel(x_ref, scale_ref, w_ref, b_ref, cos_ref, sin_ref,
                q_ref, k_ref, v_ref):
    x = x_ref[...]
    ms = jnp.mean(x * x, axis=-1, keepdims=True)
    xn = x * jax.lax.rsqrt(ms + EPS) * scale_ref[...]
    xl = jnp.roll(xn, -1, axis=1)
    xr = jnp.roll(xn, 1, axis=1)
    j = jax.lax.broadcasted_iota(jnp.int32, xn.shape, 1)
    y = jnp.where(j % 2 == 0, -xl, xr)
    roped = xn * cos_ref[...] + y * sin_ref[...]
    w = w_ref[...]
    qk = jax.lax.dot_general(roped, w[: 2 * D, :],
                             (((1,), (1,)), ((), ())),
                             preferred_element_type=jnp.float32)
    vv = jax.lax.dot_general(xn, w[2 * D:, :],
                             (((1,), (1,)), ((), ())),
                             preferred_element_type=jnp.float32)
    b = b_ref[...]
    q_ref[...] = qk[:, :D] + b[:, :D]
    k_ref[...] = qk[:, D:] + b[:, D: 2 * D]
    v_ref[...] = vv + b[:, 2 * D:]


def _attn_kernel(q_ref, k_ref, v_ref, o_ref):
    qi = pl.program_id(1)
    qs = qi * BT
    w0 = jnp.maximum(qs - WIN, 0)
    q = q_ref[0]
    kwin = k_ref[0, pl.ds(w0, KW), :]
    vwin = v_ref[0, pl.ds(w0, KW), :]
    s = jax.lax.dot_general(q, kwin, (((1,), (1,)), ((), ())),
                            preferred_element_type=jnp.float32)
    s = s * (1.0 / float(np.sqrt(HD)))
    i = qs + jax.lax.broadcasted_iota(jnp.int32, s.shape, 0)
    jj = w0 + jax.lax.broadcasted_iota(jnp.int32, s.shape, 1)
    mask = (jj <= i) & (jj > i - WIN)
    s = jnp.where(mask, s, -jnp.inf)
    m = jnp.max(s, axis=-1, keepdims=True)
    p = jnp.exp(s - m)
    num = jnp.dot(p, vwin, preferred_element_type=jnp.float32)
    o_ref[0] = num / jnp.sum(p, axis=-1, keepdims=True)


def _post_kernel(ao_ref, wo_ref, bo_ref, scale_ref, rw_ref, rb_ref,
                 xatt_ref, xm_ref, gates_ref):
    xatt = jax.lax.dot_general(ao_ref[...], wo_ref[...],
                               (((1,), (1,)), ((), ())),
                               preferred_element_type=jnp.float32) + bo_ref[...]
    xatt_ref[...] = xatt
    ms = jnp.mean(xatt * xatt, axis=-1, keepdims=True)
    xm = xatt * jax.lax.rsqrt(ms + EPS) * scale_ref[...]
    xm_ref[...] = xm
    logits = jax.lax.dot_general(xm, rw_ref[...], (((1,), (1,)), ((), ())),
                                 preferred_element_type=jnp.float32) + rb_ref[...]
    e_iota = jax.lax.broadcasted_iota(jnp.int32, logits.shape, 1)
    m1 = jnp.max(logits, axis=-1, keepdims=True)
    is1 = logits == m1
    a1 = jnp.min(jnp.where(is1, e_iota, E), axis=-1, keepdims=True)
    l2 = jnp.where(e_iota == a1, -jnp.inf, logits)
    m2 = jnp.max(l2, axis=-1, keepdims=True)
    is2 = l2 == m2
    a2 = jnp.min(jnp.where(is2, e_iota, E), axis=-1, keepdims=True)
    sel = (e_iota == a1) | (e_iota == a2)
    ex = jnp.where(sel, jnp.exp(logits - m1), 0.0)
    gates_ref[...] = ex / jnp.sum(ex, axis=-1, keepdims=True)


def _moe_kernel(xm_ref, xatt_ref, g_ref, w1_ref, b1_ref, w2_ref, b2_ref,
                out_ref):
    e = pl.program_id(0)
    c = pl.program_id(1)

    @pl.when((e == 0) & (c == 0))
    def _init():
        out_ref[...] = xatt_ref[...]

    xm = xm_ref[...]
    h = jax.lax.dot_general(xm, w1_ref[0], (((1,), (1,)), ((), ())),
                            preferred_element_type=jnp.float32) + b1_ref[0]
    h = h * jax.nn.sigmoid(h)
    o = jax.lax.dot_general(h, w2_ref[0], (((1,), (1,)), ((), ())),
                            preferred_element_type=jnp.float32)

    @pl.when(c == 0)
    def _bias():
        out_ref[...] = out_ref[...] + g_ref[0] * b2_ref[0]

    out_ref[...] = out_ref[...] + g_ref[0] * o


def kernel(x, rms_scale, in_proj_w, in_proj_b, out_proj_w, out_proj_b,
           router_w, router_b, w1, b1, w2, b2):
    x2 = x.reshape(T, D)
    scale2 = rms_scale.reshape(1, D)
    b3 = in_proj_b.reshape(1, 3 * D)
    cos_t, sin_t = _rope_tables()

    q, k, v = pl.pallas_call(
        _qkv_kernel,
        grid=(T // BT,),
        in_specs=[
            pl.BlockSpec((BT, D), lambda i: (i, 0)),
            pl.BlockSpec((1, D), lambda i: (0, 0)),
            pl.BlockSpec((3 * D, D), lambda i: (0, 0)),
            pl.BlockSpec((1, 3 * D), lambda i: (0, 0)),
            pl.BlockSpec((BT, D), lambda i: (i, 0)),
            pl.BlockSpec((BT, D), lambda i: (i, 0)),
        ],
        out_specs=[
            pl.BlockSpec((BT, D), lambda i: (i, 0)),
            pl.BlockSpec((BT, D), lambda i: (i, 0)),
            pl.BlockSpec((BT, D), lambda i: (i, 0)),
        ],
        out_shape=[jax.ShapeDtypeStruct((T, D), jnp.float32)] * 3,
    )(x2, scale2, in_proj_w, b3, cos_t, sin_t)

    q3 = q.reshape(T, H, HD).transpose(1, 0, 2)
    k3 = k.reshape(T, H, HD).transpose(1, 0, 2)
    v3 = v.reshape(T, H, HD).transpose(1, 0, 2)

    ao3 = pl.pallas_call(
        _attn_kernel,
        grid=(H, T // BT),
        in_specs=[
            pl.BlockSpec((1, BT, HD), lambda h, i: (h, i, 0)),
            pl.BlockSpec((1, T, HD), lambda h, i: (h, 0, 0)),
            pl.BlockSpec((1, T, HD), lambda h, i: (h, 0, 0)),
        ],
        out_specs=pl.BlockSpec((1, BT, HD), lambda h, i: (h, i, 0)),
        out_shape=jax.ShapeDtypeStruct((H, T, HD), jnp.float32),
    )(q3, k3, v3)

    ao = ao3.transpose(1, 0, 2).reshape(T, D)

    xatt, xm, gates = pl.pallas_call(
        _post_kernel,
        grid=(T // BT,),
        in_specs=[
            pl.BlockSpec((BT, D), lambda i: (i, 0)),
            pl.BlockSpec((D, D), lambda i: (0, 0)),
            pl.BlockSpec((1, D), lambda i: (0, 0)),
            pl.BlockSpec((1, D), lambda i: (0, 0)),
            pl.BlockSpec((E, D), lambda i: (0, 0)),
            pl.BlockSpec((1, E), lambda i: (0, 0)),
        ],
        out_specs=[
            pl.BlockSpec((BT, D), lambda i: (i, 0)),
            pl.BlockSpec((BT, D), lambda i: (i, 0)),
            pl.BlockSpec((BT, E), lambda i: (i, 0)),
        ],
        out_shape=[
            jax.ShapeDtypeStruct((T, D), jnp.float32),
            jax.ShapeDtypeStruct((T, D), jnp.float32),
            jax.ShapeDtypeStruct((T, E), jnp.float32),
        ],
    )(ao, out_proj_w, out_proj_b.reshape(1, D), scale2,
      router_w, router_b.reshape(1, E))

    gt = gates.T.reshape(E, T, 1)
    b1r = b1.reshape(E, 1, DFF)
    b2r = b2.reshape(E, 1, D)
    DC = 1024

    out = pl.pallas_call(
        _moe_kernel,
        grid=(E, DFF // DC),
        in_specs=[
            pl.BlockSpec((T, D), lambda e, c: (0, 0)),
            pl.BlockSpec((T, D), lambda e, c: (0, 0)),
            pl.BlockSpec((1, T, 1), lambda e, c: (e, 0, 0)),
            pl.BlockSpec((1, DC, D), lambda e, c: (e, c, 0)),
            pl.BlockSpec((1, 1, DC), lambda e, c: (e, 0, c)),
            pl.BlockSpec((1, D, DC), lambda e, c: (e, 0, c)),
            pl.BlockSpec((1, 1, D), lambda e, c: (e, 0, 0)),
        ],
        out_specs=pl.BlockSpec((T, D), lambda e, c: (0, 0)),
        out_shape=jax.ShapeDtypeStruct((T, D), jnp.float32),
    )(xm, xatt, gt, w1, b1r, w2, b2r)

    return out.reshape(B, T, D)
